# baseline (speedup 1.0000x reference)
import functools

import jax
import jax.numpy as jnp
import numpy as np
from jax.experimental import pallas as pl
from jax.experimental.pallas import tpu as pltpu
from jax.experimental.pallas import tpu_sc as plsc

B, L, N, K, T, A, TOPK, H = 1024, 20, 100000, 32, 2048, 128, 32, 2
M = B * L
NIDX = T * A
GW = 128
R = 128
NEG_INF = float("-inf")


@jax.jit
def _sc_gather(item_emb_pad, ids):
    mesh = plsc.VectorSubcoreMesh(core_axis_name="c", subcore_axis_name="s")

    @functools.partial(
        pl.kernel,
        out_type=jax.ShapeDtypeStruct((NIDX, 128), jnp.float32),
        mesh=mesh,
    )
    def kern(emb_hbm, ids_hbm, out_hbm):
        def body(i_vmem, o_vmem):
            pltpu.sync_copy(emb_hbm.at[i_vmem.at[0]], o_vmem)

        pltpu.emit_pipeline(
            body,
            grid=(NIDX // GW,),
            in_specs=[pl.BlockSpec((1, GW), index_map=lambda i: (0, i))],
            out_specs=[pl.BlockSpec((GW, 128), index_map=lambda i: (i, 0))],
            core_axis_name=("c", "s"),
            dimension_semantics=(pltpu.PARALLEL,),
        )(ids_hbm, out_hbm)

    return kern(item_emb_pad, ids)


def _transpose_body(src_ref, dst_ref):
    e = src_ref[...][:, :, :K]
    dst_ref[...] = jnp.swapaxes(e, 1, 2)


def _transpose_table(embs_raw):
    Tb = 16
    return pl.pallas_call(
        _transpose_body,
        grid=(T // Tb,),
        in_specs=[pl.BlockSpec((Tb, A, 128), lambda i: (i, 0, 0))],
        out_specs=pl.BlockSpec((Tb, K, A), lambda i: (i, 0, 0)),
        out_shape=jax.ShapeDtypeStruct((T, K, A), jnp.float32),
        compiler_params=pltpu.CompilerParams(
            dimension_semantics=("parallel",)),
    )(embs_raw.reshape(T, A, 128))


def _topk_body(xtsy_sref, featsT_ref, embsT_ref, inds_ref, seqs_ref,
               emb_vmem, sem):
    i = pl.program_id(0)

    @pl.when((i == 0) | (i == (M // R) // 2))
    def _():
        pltpu.make_async_copy(embsT_ref, emb_vmem, sem).start()
        pltpu.make_async_copy(embsT_ref, emb_vmem, sem).wait()

    rows = []
    for r in range(R):
        t = xtsy_sref[i * R + r]
        et = emb_vmem[t]
        ebf = et.astype(jnp.bfloat16).astype(jnp.float32)
        fbf = featsT_ref[:, r : r + 1].astype(jnp.float32)
        prod = ebf * fbf
        h16 = prod[:16, :] + prod[16:, :]
        h8 = h16[:8, :] + h16[8:, :]
        rows.append(jnp.sum(h8, axis=0, keepdims=True))
    scores = jnp.concatenate(rows, axis=0)

    lane = jax.lax.broadcasted_iota(jnp.int32, (R, A), 1)
    cols = []
    for _ in range(TOPK):
        m = jnp.max(scores, axis=1, keepdims=True)
        idx = jnp.min(jnp.where(scores == m, lane, A), axis=1, keepdims=True)
        cols.append(idx)
        scores = jnp.where(lane == idx, NEG_INF, scores)
    inds_mat = jnp.concatenate(cols, axis=1)
    inds_ref[...] = inds_mat

    sub = jax.lax.broadcasted_iota(jnp.int32, (A, TOPK), 0)
    for r in range(R):
        t = xtsy_sref[i * R + r]
        et = emb_vmem[t]
        onehot = (sub == inds_mat[r : r + 1, :]).astype(jnp.float32)
        seq_t = jnp.dot(et, onehot, preferred_element_type=jnp.float32)
        seqs_ref[r] = jnp.swapaxes(seq_t, 0, 1)


def _scores_topk(xtsy_flat, featsT_bf, embsT):
    grid_spec = pltpu.PrefetchScalarGridSpec(
        num_scalar_prefetch=1,
        grid=(M // R,),
        in_specs=[
            pl.BlockSpec((K, R), lambda i, xs: (0, i)),
            pl.BlockSpec(memory_space=pl.ANY),
        ],
        out_specs=[
            pl.BlockSpec((R, TOPK), lambda i, xs: (i, 0)),
            pl.BlockSpec((R, TOPK, K), lambda i, xs: (i, 0, 0)),
        ],
        scratch_shapes=[
            pltpu.VMEM((T, K, A), jnp.float32),
            pltpu.SemaphoreType.DMA,
        ],
    )
    return pl.pallas_call(
        _topk_body,
        grid_spec=grid_spec,
        out_shape=[
            jax.ShapeDtypeStruct((M, TOPK), jnp.int32),
            jax.ShapeDtypeStruct((M, TOPK, K), jnp.float32),
        ],
        compiler_params=pltpu.CompilerParams(
            dimension_semantics=("parallel",)),
    )(xtsy_flat, featsT_bf, embsT)


RA = 128


def _att_body(seqs_ref, gcat_ref, ucat_ref, validf_ref, out_ref):
    gcat = gcat_ref[...]
    ucat = ucat_ref[...]
    s = seqs_ref[...]
    s2 = s.reshape(RA * TOPK, K)
    p = jnp.dot(s2, gcat, preferred_element_type=jnp.float32)
    v2 = jnp.dot(s2, ucat, preferred_element_type=jnp.float32)
    p3 = p.reshape(RA, TOPK, H * K)
    v3 = v2.reshape(RA, TOPK, H * K)

    lrows = []
    for r in range(RA):
        pr = p3[r]
        pstk = jnp.concatenate([pr[:, :K], pr[:, K:]], axis=0)
        st = jnp.swapaxes(s[r], 0, 1)
        lrows.append(jnp.dot(pstk, st, preferred_element_type=jnp.float32))
    logits = jnp.stack(lrows, axis=0)

    m = jnp.max(logits, axis=2, keepdims=True)
    e = jnp.exp(logits - m)
    att = e / jnp.sum(e, axis=2, keepdims=True)

    orows = []
    for r in range(RA):
        ar = att[r]
        acat = jnp.concatenate([ar[:TOPK, :], ar[TOPK:, :]], axis=1)
        vr = v3[r]
        vstk = jnp.concatenate([vr[:, :K], vr[:, K:]], axis=0)
        orows.append(jnp.dot(acat, vstk, preferred_element_type=jnp.float32))
    o = jnp.stack(orows, axis=0)
    out_ref[...] = (o + s) * validf_ref[...][:, :, None]


def _attention(seqs, gcat, ucat, validf):
    return pl.pallas_call(
        _att_body,
        grid=(M // RA,),
        in_specs=[
            pl.BlockSpec((RA, TOPK, K), lambda i: (i, 0, 0)),
            pl.BlockSpec((K, H * K), lambda i: (0, 0)),
            pl.BlockSpec((K, H * K), lambda i: (0, 0)),
            pl.BlockSpec((RA, 1), lambda i: (i, 0)),
        ],
        out_specs=pl.BlockSpec((RA, TOPK, K), lambda i: (i, 0, 0)),
        out_shape=jax.ShapeDtypeStruct((M, TOPK, K), jnp.float32),
        compiler_params=pltpu.CompilerParams(
            dimension_semantics=("parallel",)),
    )(seqs, gcat, ucat, validf)


def kernel(inputs, xtsy, av_tens, feats, item_emb, Wq, Wk, Wv, Wo):
    flat_xtsy = xtsy.reshape(-1)
    item_emb_pad = jnp.pad(item_emb, ((0, 0), (0, 128 - K)))
    embs_raw = _sc_gather(item_emb_pad, av_tens.reshape(1, NIDX))
    embsT = _transpose_table(embs_raw)

    featsT_bf = feats.reshape(M, K).T.astype(jnp.bfloat16)
    inds, seqs = _scores_topk(flat_xtsy, featsT_bf, embsT)

    dh = K // H
    scale = 1.0 / np.sqrt(dh)
    gs, us = [], []
    for h in range(H):
        wq_h = Wq[:, h * dh : (h + 1) * dh]
        wk_h = Wk[:, h * dh : (h + 1) * dh]
        wv_h = Wv[:, h * dh : (h + 1) * dh]
        wo_h = Wo[h * dh : (h + 1) * dh, :]
        gs.append((wq_h @ wk_h.T) * scale)
        us.append(wv_h @ wo_h)
    gcat = jnp.concatenate(gs, axis=1)
    ucat = jnp.concatenate(us, axis=1)

    valid = (inputs.reshape(-1) != 0)
    validf = valid.astype(jnp.float32)[:, None]
    out = _attention(seqs, gcat, ucat, validf).reshape(B, L, TOPK, K)
    batch_inds = (inds * valid[:, None]).reshape(B, L, TOPK)
    return out, batch_inds


# speedup vs baseline: 1.0677x; 1.0677x over previous
import functools

import jax
import jax.numpy as jnp
import numpy as np
from jax.experimental import pallas as pl
from jax.experimental.pallas import tpu as pltpu
from jax.experimental.pallas import tpu_sc as plsc

B, L, N, K, T, A, TOPK, H = 1024, 20, 100000, 32, 2048, 128, 32, 2
M = B * L
NIDX = T * A
GW = 128
R = 128
NEG_INF = float("-inf")


@functools.partial(jax.jit, static_argnums=2)
def _sc_gather(src, ids, nrows):
    mesh = plsc.VectorSubcoreMesh(core_axis_name="c", subcore_axis_name="s")

    @functools.partial(
        pl.kernel,
        out_type=jax.ShapeDtypeStruct((nrows, 128), jnp.float32),
        mesh=mesh,
    )
    def kern(emb_hbm, ids_hbm, out_hbm):
        def body(i_vmem, o_vmem):
            pltpu.sync_copy(emb_hbm.at[i_vmem.at[0]], o_vmem)

        pltpu.emit_pipeline(
            body,
            grid=(nrows // GW,),
            in_specs=[pl.BlockSpec((1, GW), index_map=lambda i: (0, i))],
            out_specs=[pl.BlockSpec((GW, 128), index_map=lambda i: (i, 0))],
            core_axis_name=("c", "s"),
            dimension_semantics=(pltpu.PARALLEL,),
        )(ids_hbm, out_hbm)

    return kern(src, ids)


def _transpose_body(src_ref, dst_ref):
    e = src_ref[...][:, :, :K]
    dst_ref[...] = jnp.swapaxes(e, 1, 2)


def _transpose_table(embs_raw):
    Tb = 16
    return pl.pallas_call(
        _transpose_body,
        grid=(T // Tb,),
        in_specs=[pl.BlockSpec((Tb, A, 128), lambda i: (i, 0, 0))],
        out_specs=pl.BlockSpec((Tb, K, A), lambda i: (i, 0, 0)),
        out_shape=jax.ShapeDtypeStruct((T, K, A), jnp.float32),
        compiler_params=pltpu.CompilerParams(
            dimension_semantics=("parallel",)),
    )(embs_raw.reshape(T, A, 128))


def _topk_body(xtsy_sref, featsT_ref, embsT_ref, inds_ref, emb_vmem, sem):
    i = pl.program_id(0)

    @pl.when((i == 0) | (i == (M // R) // 2))
    def _():
        pltpu.make_async_copy(embsT_ref, emb_vmem, sem).start()
        pltpu.make_async_copy(embsT_ref, emb_vmem, sem).wait()

    rows = []
    for r in range(R):
        t = xtsy_sref[i * R + r]
        et = emb_vmem[t]
        ebf = et.astype(jnp.bfloat16).astype(jnp.float32)
        fbf = featsT_ref[:, r : r + 1].astype(jnp.float32)
        prod = ebf * fbf
        h16 = prod[:16, :] + prod[16:, :]
        h8 = h16[:8, :] + h16[8:, :]
        rows.append(jnp.sum(h8, axis=0, keepdims=True))
    NC = 4
    RC = R // NC
    lane = jax.lax.broadcasted_iota(jnp.int32, (RC, A), 1)
    chunks = [jnp.concatenate(rows[c * RC : (c + 1) * RC], axis=0)
              for c in range(NC)]
    cols = [[] for _ in range(NC)]
    for _ in range(TOPK):
        for c in range(NC):
            s_c = chunks[c]
            m = jnp.max(s_c, axis=1, keepdims=True)
            idx = jnp.min(jnp.where(s_c == m, lane, A), axis=1, keepdims=True)
            cols[c].append(idx)
            chunks[c] = jnp.where(lane == idx, NEG_INF, s_c)
    inds_mat = jnp.concatenate(
        [jnp.concatenate(cols[c], axis=1) for c in range(NC)], axis=0)
    inds_ref[...] = inds_mat


def _scores_topk(xtsy_flat, featsT_bf, embsT):
    grid_spec = pltpu.PrefetchScalarGridSpec(
        num_scalar_prefetch=1,
        grid=(M // R,),
        in_specs=[
            pl.BlockSpec((K, R), lambda i, xs: (0, i)),
            pl.BlockSpec(memory_space=pl.ANY),
        ],
        out_specs=pl.BlockSpec((R, TOPK), lambda i, xs: (i, 0)),
        scratch_shapes=[
            pltpu.VMEM((T, K, A), jnp.float32),
            pltpu.SemaphoreType.DMA,
        ],
    )
    return pl.pallas_call(
        _topk_body,
        grid_spec=grid_spec,
        out_shape=jax.ShapeDtypeStruct((M, TOPK), jnp.int32),
        compiler_params=pltpu.CompilerParams(
            dimension_semantics=("parallel",)),
    )(xtsy_flat, featsT_bf, embsT)


RA = 128


def _att_body(seqs_ref, gcat_ref, ucat_ref, validf_ref, out_ref):
    gcat = gcat_ref[...]
    ucat = ucat_ref[...]
    s = seqs_ref[...][:, :, :K]
    s2 = s.reshape(RA * TOPK, K)
    p = jnp.dot(s2, gcat, preferred_element_type=jnp.float32)
    v2 = jnp.dot(s2, ucat, preferred_element_type=jnp.float32)
    p3 = p.reshape(RA, TOPK, H * K)
    v3 = v2.reshape(RA, TOPK, H * K)

    lrows = []
    for r in range(RA):
        pr = p3[r]
        pstk = jnp.concatenate([pr[:, :K], pr[:, K:]], axis=0)
        st = jnp.swapaxes(s[r], 0, 1)
        lrows.append(jnp.dot(pstk, st, preferred_element_type=jnp.float32))
    logits = jnp.stack(lrows, axis=0)

    m = jnp.max(logits, axis=2, keepdims=True)
    e = jnp.exp(logits - m)
    att = e / jnp.sum(e, axis=2, keepdims=True)

    orows = []
    for r in range(RA):
        ar = att[r]
        acat = jnp.concatenate([ar[:TOPK, :], ar[TOPK:, :]], axis=1)
        vr = v3[r]
        vstk = jnp.concatenate([vr[:, :K], vr[:, K:]], axis=0)
        orows.append(jnp.dot(acat, vstk, preferred_element_type=jnp.float32))
    o = jnp.stack(orows, axis=0)
    out_ref[...] = (o + s) * validf_ref[...][:, :, None]


def _attention(seqs, gcat, ucat, validf):
    return pl.pallas_call(
        _att_body,
        grid=(M // RA,),
        in_specs=[
            pl.BlockSpec((RA, TOPK, 128), lambda i: (i, 0, 0)),
            pl.BlockSpec((K, H * K), lambda i: (0, 0)),
            pl.BlockSpec((K, H * K), lambda i: (0, 0)),
            pl.BlockSpec((RA, 1), lambda i: (i, 0)),
        ],
        out_specs=pl.BlockSpec((RA, TOPK, K), lambda i: (i, 0, 0)),
        out_shape=jax.ShapeDtypeStruct((M, TOPK, K), jnp.float32),
        compiler_params=pltpu.CompilerParams(
            dimension_semantics=("parallel",)),
    )(seqs, gcat, ucat, validf)


def kernel(inputs, xtsy, av_tens, feats, item_emb, Wq, Wk, Wv, Wo):
    flat_xtsy = xtsy.reshape(-1)
    item_emb_pad = jnp.pad(item_emb, ((0, 0), (0, 128 - K)))
    embs_raw = _sc_gather(item_emb_pad, av_tens.reshape(1, NIDX), NIDX)
    embsT = _transpose_table(embs_raw)

    featsT_bf = feats.reshape(M, K).T.astype(jnp.bfloat16)
    inds = _scores_topk(flat_xtsy, featsT_bf, embsT)
    gsel = (flat_xtsy[:, None] * A + inds).reshape(1, M * TOPK)
    seqs = _sc_gather(embs_raw, gsel, M * TOPK).reshape(M, TOPK, 128)

    dh = K // H
    scale = 1.0 / np.sqrt(dh)
    gs, us = [], []
    for h in range(H):
        wq_h = Wq[:, h * dh : (h + 1) * dh]
        wk_h = Wk[:, h * dh : (h + 1) * dh]
        wv_h = Wv[:, h * dh : (h + 1) * dh]
        wo_h = Wo[h * dh : (h + 1) * dh, :]
        gs.append((wq_h @ wk_h.T) * scale)
        us.append(wv_h @ wo_h)
    gcat = jnp.concatenate(gs, axis=1)
    ucat = jnp.concatenate(us, axis=1)

    valid = (inputs.reshape(-1) != 0)
    validf = valid.astype(jnp.float32)[:, None]
    out = _attention(seqs, gcat, ucat, validf).reshape(B, L, TOPK, K)
    batch_inds = (inds * valid[:, None]).reshape(B, L, TOPK)
    return out, batch_inds
